# Initial kernel scaffold; baseline (speedup 1.0000x reference)
import jax
import jax.numpy as jnp
from jax.experimental import pallas as pl


def kernel(inp_mu, inp_sc, cls_idx, cls_mu_queue, cls_sc_queue):
    raise NotImplementedError("write your pallas kernel here")


# speedup vs baseline: 1.1997x; 1.1997x over previous
import jax
import jax.numpy as jnp
from jax.experimental import pallas as pl
from jax.experimental.pallas import tpu as pltpu


BATCH = 256


def _dotT(a, b, precision):
    return jax.lax.dot_general(
        a, b, (((0,), (0,)), ((), ())),
        preferred_element_type=jnp.float32, precision=precision)


def _topk_kernel(cls_idx_col_ref, inp_sc_ref, cls_sc_queue_ref,
                 new_sc_ref, top_idx_ref):
    n_class = cls_sc_queue_ref.shape[0]
    c = cls_idx_col_ref.shape[0]
    n_mu = cls_sc_queue_ref.shape[1]
    batch = inp_sc_ref.shape[0]
    hi = jax.lax.Precision.HIGHEST

    lane_n = jax.lax.broadcasted_iota(jnp.int32, (c, n_class), 1)
    oh = (lane_n == cls_idx_col_ref[...]).astype(jnp.float32)

    sc_q = jax.lax.dot_general(
        oh, cls_sc_queue_ref[...], (((1,), (0,)), ((), ())),
        preferred_element_type=jnp.float32, precision=hi)
    inp_sel = jax.lax.dot_general(
        oh, inp_sc_ref[...], (((1,), (1,)), ((), ())),
        preferred_element_type=jnp.float32, precision=hi)

    scores = jnp.concatenate([sc_q, inp_sel], axis=1)
    n_entries = n_mu + batch
    iota_e = jax.lax.broadcasted_iota(jnp.int32, (c, n_entries), 1)

    ms, idxs = [], []
    for _ in range(n_mu):
        m = jnp.max(scores, axis=1, keepdims=True)
        cand = jnp.where(scores == m, iota_e, n_entries)
        idx = jnp.min(cand, axis=1, keepdims=True)
        ms.append(m)
        idxs.append(idx)
        scores = jnp.where(iota_e == idx, -jnp.inf, scores)

    sorted_sc = jnp.concatenate(ms, axis=1)
    top_idx_ref[...] = jnp.concatenate(idxs, axis=1)

    update = _dotT(oh, sorted_sc, hi)
    touched = _dotT(oh, jnp.ones((c, 1), jnp.float32), hi)
    new_sc_ref[...] = jnp.where(touched > 0.5, update,
                                cls_sc_queue_ref[...])


def _scatter_mu_kernel(cls_idx_ref, top_idx_ref, inp_mu_ref,
                       mu_q_ref, out_ref):
    del cls_idx_ref
    i = pl.program_id(0)
    n_mu = mu_q_ref.shape[1]
    batch = inp_mu_ref.shape[0]
    hi = jax.lax.Precision.HIGHEST

    idx_row = top_idx_ref[pl.ds(i, 1), :]
    sub_q = jax.lax.broadcasted_iota(jnp.int32, (n_mu, n_mu), 0)
    sub_b = jax.lax.broadcasted_iota(jnp.int32, (batch, n_mu), 0)
    oh_q = (sub_q == idx_row).astype(jnp.float32)
    oh_b = (sub_b == (idx_row - n_mu)).astype(jnp.float32)

    out = _dotT(oh_q, mu_q_ref[0], hi) + _dotT(oh_b, inp_mu_ref[...], hi)
    out_ref[0] = out


@jax.jit
def kernel(inp_mu, inp_sc, cls_idx, cls_mu_queue, cls_sc_queue):
    n_class, n_mu, d = cls_mu_queue.shape
    c = cls_idx.shape[0]

    new_sc_queue, top_idx = pl.pallas_call(
        _topk_kernel,
        out_shape=(
            jax.ShapeDtypeStruct((n_class, n_mu), jnp.float32),
            jax.ShapeDtypeStruct((c, n_mu), jnp.int32),
        ),
    )(cls_idx.reshape(c, 1), inp_sc, cls_sc_queue)

    grid_spec = pltpu.PrefetchScalarGridSpec(
        num_scalar_prefetch=1,
        grid=(c,),
        in_specs=[
            pl.BlockSpec((c, n_mu), lambda i, sp: (0, 0)),
            pl.BlockSpec((BATCH, d), lambda i, sp: (0, 0)),
            pl.BlockSpec((1, n_mu, d), lambda i, sp: (sp[i], 0, 0)),
        ],
        out_specs=pl.BlockSpec((1, n_mu, d), lambda i, sp: (sp[i], 0, 0)),
    )
    new_mu_queue = pl.pallas_call(
        _scatter_mu_kernel,
        grid_spec=grid_spec,
        out_shape=jax.ShapeDtypeStruct((n_class, n_mu, d), jnp.float32),
        input_output_aliases={3: 0},
    )(cls_idx, top_idx, inp_mu, cls_mu_queue)

    return new_mu_queue, new_sc_queue
